# baseline (speedup 1.0000x reference)
import functools

import jax
import jax.numpy as jnp
from jax.experimental import pallas as pl
from jax.experimental.pallas import tpu as pltpu

H = 2048
HEADS = 16
NOPE = 128
ROPE = 64
VD = 128
QHD = NOPE + ROPE
E = 8
TOPK = 2
INTER = 1408
SH = 2 * INTER
T = 2048
EPS = 1e-6
SCALE = QHD ** (-0.5)

BT = 128
P = T * TOPK
P_PAD = P + E * BT
NT = P_PAD // BT

F32 = jnp.float32
BF16 = jnp.bfloat16


def _norm_body(x_ref, w_ref, o_ref):
    x = x_ref[...]
    var = jnp.mean(x * x, axis=-1, keepdims=True)
    o_ref[...] = (x * jax.lax.rsqrt(var + EPS) * w_ref[...]).astype(BF16)


def _rmsnorm_bf16(x, w):
    return pl.pallas_call(
        _norm_body,
        grid=(8,),
        in_specs=[
            pl.BlockSpec((T // 8, H), lambda i: (i, 0)),
            pl.BlockSpec((1, H), lambda i: (0, 0)),
        ],
        out_specs=pl.BlockSpec((T // 8, H), lambda i: (i, 0)),
        out_shape=jax.ShapeDtypeStruct((T, H), BF16),
    )(x, w.reshape(1, H))


def _norm2_body(x_ref, w_ref, g_ref, o_ref, l_ref):
    x = x_ref[...]
    var = jnp.mean(x * x, axis=-1, keepdims=True)
    xm = (x * jax.lax.rsqrt(var + EPS) * w_ref[...]).astype(BF16)
    o_ref[...] = xm
    l_ref[...] = jax.lax.dot_general(
        xm, g_ref[...], (((1,), (1,)), ((), ())), preferred_element_type=F32)


def _rmsnorm_gate(x, w, gate_bf):
    return pl.pallas_call(
        _norm2_body,
        grid=(8,),
        in_specs=[
            pl.BlockSpec((T // 8, H), lambda i: (i, 0)),
            pl.BlockSpec((1, H), lambda i: (0, 0)),
            pl.BlockSpec((E, H), lambda i: (0, 0)),
        ],
        out_specs=[
            pl.BlockSpec((T // 8, H), lambda i: (i, 0)),
            pl.BlockSpec((T // 8, E), lambda i: (i, 0)),
        ],
        out_shape=[
            jax.ShapeDtypeStruct((T, H), BF16),
            jax.ShapeDtypeStruct((T, E), F32),
        ],
    )(x, w.reshape(1, H), gate_bf)


def _mm_nt_body(x_ref, w_ref, o_ref):
    o_ref[...] = jax.lax.dot_general(
        x_ref[...], w_ref[...], (((1,), (1,)), ((), ())),
        preferred_element_type=F32).astype(o_ref.dtype)


def _qkv_proj(hn_bf, w_bf):
    n_out = w_bf.shape[0]
    bn = 512
    return pl.pallas_call(
        _mm_nt_body,
        grid=(n_out // bn,),
        in_specs=[
            pl.BlockSpec((T, H), lambda n: (0, 0)),
            pl.BlockSpec((bn, H), lambda n: (n, 0)),
        ],
        out_specs=pl.BlockSpec((T, bn), lambda n: (0, n)),
        out_shape=jax.ShapeDtypeStruct((T, n_out), BF16),
    )(hn_bf, w_bf)


def _resid_mm_body(x_ref, w_ref, r_ref, o_ref):
    o_ref[...] = r_ref[...] + jax.lax.dot_general(
        x_ref[...], w_ref[...], (((1,), (1,)), ((), ())),
        preferred_element_type=F32)


def _out_proj(ctx_bf, wo_bf, resid):
    bn = 512
    return pl.pallas_call(
        _resid_mm_body,
        grid=(H // bn,),
        in_specs=[
            pl.BlockSpec((T, H), lambda n: (0, 0)),
            pl.BlockSpec((bn, H), lambda n: (n, 0)),
            pl.BlockSpec((T, bn), lambda n: (0, n)),
        ],
        out_specs=pl.BlockSpec((T, bn), lambda n: (0, n)),
        out_shape=jax.ShapeDtypeStruct((T, H), F32),
    )(ctx_bf, wo_bf, resid)


def _attn_body(q_ref, k_ref, v_ref, o_ref):
    qi = pl.program_id(1)
    s = jax.lax.dot_general(
        q_ref[...], k_ref[...], (((1,), (1,)), ((), ())),
        preferred_element_type=F32)
    row = qi * 128 + jax.lax.broadcasted_iota(jnp.int32, (128, T), 0)
    col = jax.lax.broadcasted_iota(jnp.int32, (128, T), 1)
    s = jnp.where(col <= row, s * SCALE, jnp.float32(-1e9))
    m = jnp.max(s, axis=-1, keepdims=True)
    p = jnp.exp(s - m)
    l = jnp.sum(p, axis=-1, keepdims=True)
    attn = (p / l).astype(BF16)
    o_ref[...] = jax.lax.dot_general(
        attn, v_ref[...], (((1,), (0,)), ((), ())),
        preferred_element_type=F32).astype(BF16)


def _attention(qkv_bf):
    return pl.pallas_call(
        _attn_body,
        grid=(HEADS, T // 128),
        in_specs=[
            pl.BlockSpec((128, 128), lambda h, qi: (qi, h)),
            pl.BlockSpec((T, 128), lambda h, qi: (0, HEADS + h)),
            pl.BlockSpec((T, 128), lambda h, qi: (0, 2 * HEADS + h)),
        ],
        out_specs=pl.BlockSpec((128, 128), lambda h, qi: (qi, h)),
        out_shape=jax.ShapeDtypeStruct((T, HEADS * VD), BF16),
    )(qkv_bf, qkv_bf, qkv_bf)


def _moe_body(te_ref, xs_ref, w1g_ref, w1u_ref, w2_ref, y_ref):
    xs = xs_ref[...]
    g = jax.lax.dot_general(
        xs, w1g_ref[0, 0], (((1,), (1,)), ((), ())), preferred_element_type=F32)
    u = jax.lax.dot_general(
        xs, w1u_ref[0, 0], (((1,), (1,)), ((), ())), preferred_element_type=F32)
    act = (jax.nn.silu(g) * u).astype(BF16)
    y_ref[...] = jax.lax.dot_general(
        act, w2_ref[0], (((1,), (1,)), ((), ())), preferred_element_type=F32)


def _moe_grouped(xs_bf, w1_bf, w2_bf, tile_e):
    grid_spec = pltpu.PrefetchScalarGridSpec(
        num_scalar_prefetch=1,
        grid=(NT,),
        in_specs=[
            pl.BlockSpec((BT, H), lambda i, te: (i, 0)),
            pl.BlockSpec((1, 1, INTER, H), lambda i, te: (te[i], 0, 0, 0)),
            pl.BlockSpec((1, 1, INTER, H), lambda i, te: (te[i], 1, 0, 0)),
            pl.BlockSpec((1, H, INTER), lambda i, te: (te[i], 0, 0)),
        ],
        out_specs=pl.BlockSpec((BT, H), lambda i, te: (i, 0)),
    )
    return pl.pallas_call(
        _moe_body,
        grid_spec=grid_spec,
        out_shape=jax.ShapeDtypeStruct((P_PAD, H), F32),
    )(tile_e, xs_bf, w1_bf.reshape(E, 2, INTER, H), w1_bf.reshape(E, 2, INTER, H), w2_bf)


def _shup_body(x_ref, wg_ref, wu_ref, o_ref):
    x = x_ref[...]
    g = jax.lax.dot_general(
        x, wg_ref[...], (((1,), (1,)), ((), ())), preferred_element_type=F32)
    u = jax.lax.dot_general(
        x, wu_ref[...], (((1,), (1,)), ((), ())), preferred_element_type=F32)
    o_ref[...] = (jax.nn.silu(g) * u).astype(BF16)


def _shared_up(xm_bf, shexp_bf):
    bn = INTER
    return pl.pallas_call(
        _shup_body,
        grid=(SH // bn, T // 128),
        in_specs=[
            pl.BlockSpec((128, H), lambda n, i: (i, 0)),
            pl.BlockSpec((bn, H), lambda n, i: (n, 0)),
            pl.BlockSpec((bn, H), lambda n, i: (n + SH // bn, 0)),
        ],
        out_specs=pl.BlockSpec((128, bn), lambda n, i: (i, n)),
        out_shape=jax.ShapeDtypeStruct((T, SH), BF16),
    )(xm_bf, shexp_bf, shexp_bf)


def _shdown_body(a_ref, w_ref, r_ref, o_ref):
    o_ref[...] = r_ref[...] + jax.lax.dot_general(
        a_ref[...], w_ref[...], (((1,), (1,)), ((), ())),
        preferred_element_type=F32)


def _shared_down_base(act_bf, shdown_bf, hidden):
    return pl.pallas_call(
        _shdown_body,
        grid=(T // 128,),
        in_specs=[
            pl.BlockSpec((128, SH), lambda i: (i, 0)),
            pl.BlockSpec((H, SH), lambda i: (0, 0)),
            pl.BlockSpec((128, H), lambda i: (i, 0)),
        ],
        out_specs=pl.BlockSpec((128, H), lambda i: (i, 0)),
        out_shape=jax.ShapeDtypeStruct((T, H), F32),
    )(act_bf, shdown_bf, hidden)


def _combine_body(b_ref, y0_ref, y1_ref, w0_ref, w1_ref, o_ref):
    o_ref[...] = (b_ref[...] + w0_ref[...] * y0_ref[...]
                  + w1_ref[...] * y1_ref[...])


def _combine(base, y0g, y1g, w0, w1):
    bt = 256
    return pl.pallas_call(
        _combine_body,
        grid=(T // bt,),
        in_specs=[
            pl.BlockSpec((bt, H), lambda i: (i, 0)),
            pl.BlockSpec((bt, H), lambda i: (i, 0)),
            pl.BlockSpec((bt, H), lambda i: (i, 0)),
            pl.BlockSpec((bt, 1), lambda i: (i, 0)),
            pl.BlockSpec((bt, 1), lambda i: (i, 0)),
        ],
        out_specs=pl.BlockSpec((bt, H), lambda i: (i, 0)),
        out_shape=jax.ShapeDtypeStruct((T, H), F32),
    )(base, y0g, y1g, w0.reshape(T, 1), w1.reshape(T, 1))


def _route(logits):
    probs = jax.nn.softmax(logits, axis=-1)
    topv, topi = jax.lax.top_k(probs, TOPK)
    topw = topv / jnp.sum(topv, axis=-1, keepdims=True)
    e_flat = topi.reshape(-1)
    order = jnp.argsort(e_flat, stable=True)
    sorted_e = e_flat[order]
    counts = jnp.bincount(e_flat, length=E)
    excl = jnp.cumsum(counts) - counts
    padded = ((counts + BT - 1) // BT) * BT
    poff = jnp.cumsum(padded) - padded
    within = jnp.arange(P, dtype=jnp.int32) - excl[sorted_e]
    dest = poff[sorted_e] + within
    tok_pad = jnp.zeros((P_PAD,), jnp.int32).at[dest].set(
        (order // TOPK).astype(jnp.int32))
    tile_start = jnp.arange(NT, dtype=jnp.int32) * BT
    pend = poff + padded
    tile_e = jnp.minimum(
        jnp.sum(tile_start[:, None] >= pend[None, :], axis=1), E - 1
    ).astype(jnp.int32)
    pos = jnp.zeros((P,), jnp.int32).at[order].set(dest.astype(jnp.int32))
    pos2 = pos.reshape(T, TOPK)
    return tok_pad, tile_e, pos2, topw


def kernel(hidden_states, positions, kv_cache, slot_mapping, seq_lens,
           ln1_w, ln2_w, wq, wkv, wo, gate_w, w1, w2, shexp_w, shdown_w):
    x = hidden_states.reshape(T, H)

    wq_nope = wq.reshape(HEADS, QHD, H)[:, :NOPE, :].reshape(HEADS * NOPE, H)
    w_qkv = jnp.concatenate(
        [wq_nope, wkv[:HEADS * (NOPE + VD)]], axis=0).astype(BF16)
    wo_bf = wo.astype(BF16)
    gate_bf = gate_w.astype(BF16)
    w1_bf = w1.astype(BF16)
    w2_bf = w2.astype(BF16)
    shexp_bf = shexp_w.astype(BF16)
    shdown_bf = shdown_w.astype(BF16)

    hn_bf = _rmsnorm_bf16(x, ln1_w)
    qkv = _qkv_proj(hn_bf, w_qkv)
    ctx = _attention(qkv)
    hidden = _out_proj(ctx, wo_bf, x)

    xm_bf, logits = _rmsnorm_gate(hidden, ln2_w, gate_bf)

    tok_pad, tile_e, pos2, topw = _route(logits)

    xs_bf = xm_bf[tok_pad]
    y = _moe_grouped(xs_bf, w1_bf, w2_bf, tile_e)
    y0g = y[pos2[:, 0]]
    y1g = y[pos2[:, 1]]

    act_sh = _shared_up(xm_bf, shexp_bf)
    base = _shared_down_base(act_sh, shdown_bf, hidden)

    out = _combine(base, y0g, y1g, topw[:, 0], topw[:, 1])
    return out.reshape(1, T, H)


# speedup vs baseline: 1.1042x; 1.1042x over previous
import functools

import jax
import jax.numpy as jnp
from jax.experimental import pallas as pl
from jax.experimental.pallas import tpu as pltpu

H = 2048
HEADS = 16
NOPE = 128
ROPE = 64
VD = 128
QHD = NOPE + ROPE
E = 8
TOPK = 2
INTER = 1408
SH = 2 * INTER
T = 2048
EPS = 1e-6
SCALE = QHD ** (-0.5)

BT = 256
P = T * TOPK
P_PAD = P + E * BT
NT = P_PAD // BT

F32 = jnp.float32
BF16 = jnp.bfloat16


def _norm_body(x_ref, w_ref, o_ref):
    x = x_ref[...]
    var = jnp.mean(x * x, axis=-1, keepdims=True)
    o_ref[...] = (x * jax.lax.rsqrt(var + EPS) * w_ref[...]).astype(BF16)


def _rmsnorm_bf16(x, w):
    return pl.pallas_call(
        _norm_body,
        grid=(8,),
        in_specs=[
            pl.BlockSpec((T // 8, H), lambda i: (i, 0)),
            pl.BlockSpec((1, H), lambda i: (0, 0)),
        ],
        out_specs=pl.BlockSpec((T // 8, H), lambda i: (i, 0)),
        out_shape=jax.ShapeDtypeStruct((T, H), BF16),
    )(x, w.reshape(1, H))


def _norm2_body(x_ref, w_ref, g_ref, o_ref, l_ref):
    x = x_ref[...]
    var = jnp.mean(x * x, axis=-1, keepdims=True)
    xm = (x * jax.lax.rsqrt(var + EPS) * w_ref[...]).astype(BF16)
    o_ref[...] = xm
    l_ref[...] = jax.lax.dot_general(
        xm, g_ref[...], (((1,), (1,)), ((), ())), preferred_element_type=F32)


def _rmsnorm_gate(x, w, gate_bf):
    return pl.pallas_call(
        _norm2_body,
        grid=(8,),
        in_specs=[
            pl.BlockSpec((T // 8, H), lambda i: (i, 0)),
            pl.BlockSpec((1, H), lambda i: (0, 0)),
            pl.BlockSpec((E, H), lambda i: (0, 0)),
        ],
        out_specs=[
            pl.BlockSpec((T // 8, H), lambda i: (i, 0)),
            pl.BlockSpec((T // 8, E), lambda i: (i, 0)),
        ],
        out_shape=[
            jax.ShapeDtypeStruct((T, H), BF16),
            jax.ShapeDtypeStruct((T, E), F32),
        ],
    )(x, w.reshape(1, H), gate_bf)


def _mm_nt_body(x_ref, w_ref, o_ref):
    o_ref[...] = jax.lax.dot_general(
        x_ref[...], w_ref[...], (((1,), (1,)), ((), ())),
        preferred_element_type=F32).astype(o_ref.dtype)


def _qkv_proj(hn_bf, w_bf):
    n_out = w_bf.shape[0]
    bn = 512
    return pl.pallas_call(
        _mm_nt_body,
        grid=(n_out // bn,),
        in_specs=[
            pl.BlockSpec((T, H), lambda n: (0, 0)),
            pl.BlockSpec((bn, H), lambda n: (n, 0)),
        ],
        out_specs=pl.BlockSpec((T, bn), lambda n: (0, n)),
        out_shape=jax.ShapeDtypeStruct((T, n_out), BF16),
    )(hn_bf, w_bf)


def _resid_mm_body(x_ref, w_ref, r_ref, o_ref):
    o_ref[...] = r_ref[...] + jax.lax.dot_general(
        x_ref[...], w_ref[...], (((1,), (1,)), ((), ())),
        preferred_element_type=F32)


def _out_proj(ctx_bf, wo_bf, resid):
    bn = 512
    return pl.pallas_call(
        _resid_mm_body,
        grid=(H // bn,),
        in_specs=[
            pl.BlockSpec((T, H), lambda n: (0, 0)),
            pl.BlockSpec((bn, H), lambda n: (n, 0)),
            pl.BlockSpec((T, bn), lambda n: (0, n)),
        ],
        out_specs=pl.BlockSpec((T, bn), lambda n: (0, n)),
        out_shape=jax.ShapeDtypeStruct((T, H), F32),
    )(ctx_bf, wo_bf, resid)


BQ = 256
BK = 256


def _attn_body(q_ref, k_ref, v_ref, o_ref, s_scr):
    qi = pl.program_id(1)
    q = q_ref[...]

    def p1(kt, m):
        k = k_ref[pl.ds(kt * BK, BK), :]
        s = jax.lax.dot_general(
            q, k, (((1,), (1,)), ((), ())), preferred_element_type=F32)
        row = qi * BQ + jax.lax.broadcasted_iota(jnp.int32, (BQ, BK), 0)
        col = kt * BK + jax.lax.broadcasted_iota(jnp.int32, (BQ, BK), 1)
        s = jnp.where(col <= row, s * SCALE, jnp.float32(-1e9))
        s_scr[:, pl.ds(kt * BK, BK)] = s
        return jnp.maximum(m, jnp.max(s, axis=-1, keepdims=True))

    m = jax.lax.fori_loop(
        0, qi + 1, p1, jnp.full((BQ, 1), -1e30, dtype=F32))

    def p2(kt, l):
        p = jnp.exp(s_scr[:, pl.ds(kt * BK, BK)] - m)
        s_scr[:, pl.ds(kt * BK, BK)] = p
        return l + jnp.sum(p, axis=-1, keepdims=True)

    l = jax.lax.fori_loop(0, qi + 1, p2, jnp.zeros((BQ, 1), dtype=F32))

    def p3(kt, acc):
        attn = (s_scr[:, pl.ds(kt * BK, BK)] / l).astype(BF16)
        v = v_ref[pl.ds(kt * BK, BK), :]
        return acc + jax.lax.dot_general(
            attn, v, (((1,), (0,)), ((), ())), preferred_element_type=F32)

    acc = jax.lax.fori_loop(
        0, qi + 1, p3, jnp.zeros((BQ, VD), dtype=F32))
    o_ref[...] = acc.astype(BF16)


def _attention(qkv_bf):
    return pl.pallas_call(
        _attn_body,
        grid=(HEADS, T // BQ),
        in_specs=[
            pl.BlockSpec((BQ, 128), lambda h, qi: (qi, h)),
            pl.BlockSpec((T, 128), lambda h, qi: (0, HEADS + h)),
            pl.BlockSpec((T, 128), lambda h, qi: (0, 2 * HEADS + h)),
        ],
        out_specs=pl.BlockSpec((BQ, 128), lambda h, qi: (qi, h)),
        out_shape=jax.ShapeDtypeStruct((T, HEADS * VD), BF16),
        scratch_shapes=[pltpu.VMEM((BQ, T), F32)],
    )(qkv_bf, qkv_bf, qkv_bf)


def _moe_body(te_ref, nv_ref, xs_ref, w1g_ref, w1u_ref, w2_ref, y_ref):
    i = pl.program_id(0)

    @pl.when(i < nv_ref[0])
    def _():
        xs = xs_ref[...]
        g = jax.lax.dot_general(
            xs, w1g_ref[0, 0], (((1,), (1,)), ((), ())),
            preferred_element_type=F32)
        u = jax.lax.dot_general(
            xs, w1u_ref[0, 0], (((1,), (1,)), ((), ())),
            preferred_element_type=F32)
        act = (jax.nn.silu(g) * u).astype(BF16)
        y_ref[...] = jax.lax.dot_general(
            act, w2_ref[0], (((1,), (1,)), ((), ())),
            preferred_element_type=F32)


def _moe_grouped(xs_bf, w1_bf, w2_bf, tile_e, nv):
    grid_spec = pltpu.PrefetchScalarGridSpec(
        num_scalar_prefetch=2,
        grid=(NT,),
        in_specs=[
            pl.BlockSpec((BT, H), lambda i, te, nv: (i, 0)),
            pl.BlockSpec((1, 1, INTER, H), lambda i, te, nv: (te[i], 0, 0, 0)),
            pl.BlockSpec((1, 1, INTER, H), lambda i, te, nv: (te[i], 1, 0, 0)),
            pl.BlockSpec((1, H, INTER), lambda i, te, nv: (te[i], 0, 0)),
        ],
        out_specs=pl.BlockSpec((BT, H), lambda i, te, nv: (i, 0)),
    )
    return pl.pallas_call(
        _moe_body,
        grid_spec=grid_spec,
        out_shape=jax.ShapeDtypeStruct((P_PAD, H), F32),
    )(tile_e, nv, xs_bf, w1_bf.reshape(E, 2, INTER, H),
      w1_bf.reshape(E, 2, INTER, H), w2_bf)


def _shup_body(x_ref, wg_ref, wu_ref, o_ref):
    x = x_ref[...]
    g = jax.lax.dot_general(
        x, wg_ref[...], (((1,), (1,)), ((), ())), preferred_element_type=F32)
    u = jax.lax.dot_general(
        x, wu_ref[...], (((1,), (1,)), ((), ())), preferred_element_type=F32)
    o_ref[...] = (jax.nn.silu(g) * u).astype(BF16)


def _shared_up(xm_bf, shexp_bf):
    bn = INTER
    bm = 256
    return pl.pallas_call(
        _shup_body,
        grid=(SH // bn, T // bm),
        in_specs=[
            pl.BlockSpec((bm, H), lambda n, i: (i, 0)),
            pl.BlockSpec((bn, H), lambda n, i: (n, 0)),
            pl.BlockSpec((bn, H), lambda n, i: (n + SH // bn, 0)),
        ],
        out_specs=pl.BlockSpec((bm, bn), lambda n, i: (i, n)),
        out_shape=jax.ShapeDtypeStruct((T, SH), BF16),
    )(xm_bf, shexp_bf, shexp_bf)


def _shdown_body(a_ref, w_ref, r_ref, o_ref):
    o_ref[...] = r_ref[...] + jax.lax.dot_general(
        a_ref[...], w_ref[...], (((1,), (1,)), ((), ())),
        preferred_element_type=F32)


def _shared_down_base(act_bf, shdown_bf, hidden):
    bm = 256
    return pl.pallas_call(
        _shdown_body,
        grid=(T // bm,),
        in_specs=[
            pl.BlockSpec((bm, SH), lambda i: (i, 0)),
            pl.BlockSpec((H, SH), lambda i: (0, 0)),
            pl.BlockSpec((bm, H), lambda i: (i, 0)),
        ],
        out_specs=pl.BlockSpec((bm, H), lambda i: (i, 0)),
        out_shape=jax.ShapeDtypeStruct((T, H), F32),
    )(act_bf, shdown_bf, hidden)


def _combine_body(b_ref, y0_ref, y1_ref, w0_ref, w1_ref, o_ref):
    o_ref[...] = (b_ref[...] + w0_ref[...] * y0_ref[...]
                  + w1_ref[...] * y1_ref[...])


def _combine(base, y0g, y1g, w0, w1):
    bt = 256
    return pl.pallas_call(
        _combine_body,
        grid=(T // bt,),
        in_specs=[
            pl.BlockSpec((bt, H), lambda i: (i, 0)),
            pl.BlockSpec((bt, H), lambda i: (i, 0)),
            pl.BlockSpec((bt, H), lambda i: (i, 0)),
            pl.BlockSpec((bt, 1), lambda i: (i, 0)),
            pl.BlockSpec((bt, 1), lambda i: (i, 0)),
        ],
        out_specs=pl.BlockSpec((bt, H), lambda i: (i, 0)),
        out_shape=jax.ShapeDtypeStruct((T, H), F32),
    )(base, y0g, y1g, w0.reshape(T, 1), w1.reshape(T, 1))


def _route(logits):
    probs = jax.nn.softmax(logits, axis=-1)
    topv, topi = jax.lax.top_k(probs, TOPK)
    topw = topv / jnp.sum(topv, axis=-1, keepdims=True)
    e_flat = topi.reshape(-1)
    order = jnp.argsort(e_flat, stable=True)
    sorted_e = e_flat[order]
    counts = jnp.bincount(e_flat, length=E)
    excl = jnp.cumsum(counts) - counts
    padded = ((counts + BT - 1) // BT) * BT
    poff = jnp.cumsum(padded) - padded
    within = jnp.arange(P, dtype=jnp.int32) - excl[sorted_e]
    dest = poff[sorted_e] + within
    tok_pad = jnp.zeros((P_PAD,), jnp.int32).at[dest].set(
        (order // TOPK).astype(jnp.int32))
    tile_start = jnp.arange(NT, dtype=jnp.int32) * BT
    pend = poff + padded
    nv = (jnp.sum(padded) // BT).astype(jnp.int32).reshape(1)
    tile_e_raw = jnp.minimum(
        jnp.sum(tile_start[:, None] >= pend[None, :], axis=1), E - 1
    ).astype(jnp.int32)
    tile_e = tile_e_raw[jnp.minimum(jnp.arange(NT, dtype=jnp.int32), nv[0] - 1)]
    pos = jnp.zeros((P,), jnp.int32).at[order].set(dest.astype(jnp.int32))
    pos2 = pos.reshape(T, TOPK)
    return tok_pad, tile_e, nv, pos2, topw


def kernel(hidden_states, positions, kv_cache, slot_mapping, seq_lens,
           ln1_w, ln2_w, wq, wkv, wo, gate_w, w1, w2, shexp_w, shdown_w):
    x = hidden_states.reshape(T, H)

    wq_nope = wq.reshape(HEADS, QHD, H)[:, :NOPE, :].reshape(HEADS * NOPE, H)
    w_qkv = jnp.concatenate(
        [wq_nope, wkv[:HEADS * (NOPE + VD)]], axis=0).astype(BF16)
    wo_bf = wo.astype(BF16)
    gate_bf = gate_w.astype(BF16)
    w1_bf = w1.astype(BF16)
    w2_bf = w2.astype(BF16)
    shexp_bf = shexp_w.astype(BF16)
    shdown_bf = shdown_w.astype(BF16)

    hn_bf = _rmsnorm_bf16(x, ln1_w)
    qkv = _qkv_proj(hn_bf, w_qkv)
    ctx = _attention(qkv)
    hidden = _out_proj(ctx, wo_bf, x)

    xm_bf, logits = _rmsnorm_gate(hidden, ln2_w, gate_bf)

    tok_pad, tile_e, nv, pos2, topw = _route(logits)

    xs_bf = xm_bf[tok_pad]
    y = _moe_grouped(xs_bf, w1_bf, w2_bf, tile_e, nv)
    y0g = y[pos2[:, 0]]
    y1g = y[pos2[:, 1]]

    act_sh = _shared_up(xm_bf, shexp_bf)
    base = _shared_down_base(act_sh, shdown_bf, hidden)

    out = _combine(base, y0g, y1g, topw[:, 0], topw[:, 1])
    return out.reshape(1, T, H)


# speedup vs baseline: 1.3148x; 1.1907x over previous
import functools

import jax
import jax.numpy as jnp
from jax.experimental import pallas as pl
from jax.experimental.pallas import tpu as pltpu

H = 2048
HEADS = 16
NOPE = 128
ROPE = 64
VD = 128
QHD = NOPE + ROPE
E = 8
TOPK = 2
INTER = 1408
SH = 2 * INTER
T = 2048
EPS = 1e-6
SCALE = QHD ** (-0.5)

BT = 256
P = T * TOPK
P_PAD = P + E * BT
NT = P_PAD // BT

F32 = jnp.float32
BF16 = jnp.bfloat16


def _norm_body(x_ref, w_ref, o_ref):
    x = x_ref[...]
    var = jnp.mean(x * x, axis=-1, keepdims=True)
    o_ref[...] = (x * jax.lax.rsqrt(var + EPS) * w_ref[...]).astype(BF16)


def _rmsnorm_bf16(x, w):
    return pl.pallas_call(
        _norm_body,
        grid=(8,),
        in_specs=[
            pl.BlockSpec((T // 8, H), lambda i: (i, 0)),
            pl.BlockSpec((1, H), lambda i: (0, 0)),
        ],
        out_specs=pl.BlockSpec((T // 8, H), lambda i: (i, 0)),
        out_shape=jax.ShapeDtypeStruct((T, H), BF16),
    )(x, w.reshape(1, H))


def _norm2_body(x_ref, w_ref, g_ref, o_ref, l_ref):
    x = x_ref[...]
    var = jnp.mean(x * x, axis=-1, keepdims=True)
    xm = (x * jax.lax.rsqrt(var + EPS) * w_ref[...]).astype(BF16)
    o_ref[...] = xm
    l_ref[...] = jax.lax.dot_general(
        xm, g_ref[...], (((1,), (1,)), ((), ())), preferred_element_type=F32)


def _rmsnorm_gate(x, w, gate_bf):
    return pl.pallas_call(
        _norm2_body,
        grid=(8,),
        in_specs=[
            pl.BlockSpec((T // 8, H), lambda i: (i, 0)),
            pl.BlockSpec((1, H), lambda i: (0, 0)),
            pl.BlockSpec((E, H), lambda i: (0, 0)),
        ],
        out_specs=[
            pl.BlockSpec((T // 8, H), lambda i: (i, 0)),
            pl.BlockSpec((T // 8, E), lambda i: (i, 0)),
        ],
        out_shape=[
            jax.ShapeDtypeStruct((T, H), BF16),
            jax.ShapeDtypeStruct((T, E), F32),
        ],
    )(x, w.reshape(1, H), gate_bf)


def _mm_nt_body(x_ref, w_ref, o_ref):
    o_ref[...] = jax.lax.dot_general(
        x_ref[...], w_ref[...], (((1,), (1,)), ((), ())),
        preferred_element_type=F32).astype(o_ref.dtype)


def _qkv_proj(hn_bf, w_bf):
    n_out = w_bf.shape[0]
    bn = 512
    return pl.pallas_call(
        _mm_nt_body,
        grid=(n_out // bn,),
        in_specs=[
            pl.BlockSpec((T, H), lambda n: (0, 0)),
            pl.BlockSpec((bn, H), lambda n: (n, 0)),
        ],
        out_specs=pl.BlockSpec((T, bn), lambda n: (0, n)),
        out_shape=jax.ShapeDtypeStruct((T, n_out), BF16),
    )(hn_bf, w_bf)


def _resid_mm_body(x_ref, w_ref, r_ref, o_ref):
    o_ref[...] = r_ref[...] + jax.lax.dot_general(
        x_ref[...], w_ref[...], (((1,), (1,)), ((), ())),
        preferred_element_type=F32)


def _out_proj(ctx_bf, wo_bf, resid):
    bn = 512
    return pl.pallas_call(
        _resid_mm_body,
        grid=(H // bn,),
        in_specs=[
            pl.BlockSpec((T, H), lambda n: (0, 0)),
            pl.BlockSpec((bn, H), lambda n: (n, 0)),
            pl.BlockSpec((T, bn), lambda n: (0, n)),
        ],
        out_specs=pl.BlockSpec((T, bn), lambda n: (0, n)),
        out_shape=jax.ShapeDtypeStruct((T, H), F32),
    )(ctx_bf, wo_bf, resid)


BQ = 256
BK = 256


def _attn_body(q_ref, k_ref, v_ref, o_ref):
    qi = pl.program_id(1)
    s = jax.lax.dot_general(
        q_ref[...], k_ref[...], (((1,), (1,)), ((), ())),
        preferred_element_type=F32)
    row = qi * BQ + jax.lax.broadcasted_iota(jnp.int32, (BQ, T), 0)
    col = jax.lax.broadcasted_iota(jnp.int32, (BQ, T), 1)
    s = jnp.where(col <= row, s * SCALE, jnp.float32(-1e9))
    m = jnp.max(s, axis=-1, keepdims=True)
    p = jnp.exp(s - m)
    l = jnp.sum(p, axis=-1, keepdims=True)
    attn = (p / l).astype(BF16)
    o_ref[...] = jax.lax.dot_general(
        attn, v_ref[...], (((1,), (0,)), ((), ())),
        preferred_element_type=F32).astype(BF16)


def _attention(qkv_bf):
    return pl.pallas_call(
        _attn_body,
        grid=(HEADS, T // BQ),
        in_specs=[
            pl.BlockSpec((BQ, 128), lambda h, qi: (qi, h)),
            pl.BlockSpec((T, 128), lambda h, qi: (0, HEADS + h)),
            pl.BlockSpec((T, 128), lambda h, qi: (0, 2 * HEADS + h)),
        ],
        out_specs=pl.BlockSpec((BQ, 128), lambda h, qi: (qi, h)),
        out_shape=jax.ShapeDtypeStruct((T, HEADS * VD), BF16),
    )(qkv_bf, qkv_bf, qkv_bf)


def _moe_body(te_ref, nv_ref, xs_ref, w1g_ref, w1u_ref, w2_ref, y_ref):
    i = pl.program_id(0)

    @pl.when(i < nv_ref[0])
    def _():
        xs = xs_ref[...]
        g = jax.lax.dot_general(
            xs, w1g_ref[0, 0], (((1,), (1,)), ((), ())),
            preferred_element_type=F32)
        u = jax.lax.dot_general(
            xs, w1u_ref[0, 0], (((1,), (1,)), ((), ())),
            preferred_element_type=F32)
        act = (jax.nn.silu(g) * u).astype(BF16)
        y_ref[...] = jax.lax.dot_general(
            act, w2_ref[0], (((1,), (1,)), ((), ())),
            preferred_element_type=F32)


def _moe_grouped(xs_bf, w1_bf, w2_bf, tile_e, nv):
    grid_spec = pltpu.PrefetchScalarGridSpec(
        num_scalar_prefetch=2,
        grid=(NT,),
        in_specs=[
            pl.BlockSpec((BT, H), lambda i, te, nv: (i, 0)),
            pl.BlockSpec((1, 1, INTER, H), lambda i, te, nv: (te[i], 0, 0, 0)),
            pl.BlockSpec((1, 1, INTER, H), lambda i, te, nv: (te[i], 1, 0, 0)),
            pl.BlockSpec((1, H, INTER), lambda i, te, nv: (te[i], 0, 0)),
        ],
        out_specs=pl.BlockSpec((BT, H), lambda i, te, nv: (i, 0)),
    )
    return pl.pallas_call(
        _moe_body,
        grid_spec=grid_spec,
        out_shape=jax.ShapeDtypeStruct((P_PAD, H), F32),
    )(tile_e, nv, xs_bf, w1_bf.reshape(E, 2, INTER, H),
      w1_bf.reshape(E, 2, INTER, H), w2_bf)


def _shup_body(x_ref, wg_ref, wu_ref, o_ref):
    x = x_ref[...]
    g = jax.lax.dot_general(
        x, wg_ref[...], (((1,), (1,)), ((), ())), preferred_element_type=F32)
    u = jax.lax.dot_general(
        x, wu_ref[...], (((1,), (1,)), ((), ())), preferred_element_type=F32)
    o_ref[...] = (jax.nn.silu(g) * u).astype(BF16)


def _shared_up(xm_bf, shexp_bf):
    bn = INTER
    bm = 256
    return pl.pallas_call(
        _shup_body,
        grid=(SH // bn, T // bm),
        in_specs=[
            pl.BlockSpec((bm, H), lambda n, i: (i, 0)),
            pl.BlockSpec((bn, H), lambda n, i: (n, 0)),
            pl.BlockSpec((bn, H), lambda n, i: (n + SH // bn, 0)),
        ],
        out_specs=pl.BlockSpec((bm, bn), lambda n, i: (i, n)),
        out_shape=jax.ShapeDtypeStruct((T, SH), BF16),
    )(xm_bf, shexp_bf, shexp_bf)


def _shdown_body(a_ref, w_ref, r_ref, o_ref):
    o_ref[...] = r_ref[...] + jax.lax.dot_general(
        a_ref[...], w_ref[...], (((1,), (1,)), ((), ())),
        preferred_element_type=F32)


def _shared_down_base(act_bf, shdown_bf, hidden):
    bm = 256
    return pl.pallas_call(
        _shdown_body,
        grid=(T // bm,),
        in_specs=[
            pl.BlockSpec((bm, SH), lambda i: (i, 0)),
            pl.BlockSpec((H, SH), lambda i: (0, 0)),
            pl.BlockSpec((bm, H), lambda i: (i, 0)),
        ],
        out_specs=pl.BlockSpec((bm, H), lambda i: (i, 0)),
        out_shape=jax.ShapeDtypeStruct((T, H), F32),
    )(act_bf, shdown_bf, hidden)


def _combine_body(b_ref, y0_ref, y1_ref, w0_ref, w1_ref, o_ref):
    o_ref[...] = (b_ref[...] + w0_ref[...] * y0_ref[...]
                  + w1_ref[...] * y1_ref[...])


def _combine(base, y0g, y1g, w0, w1):
    bt = 256
    return pl.pallas_call(
        _combine_body,
        grid=(T // bt,),
        in_specs=[
            pl.BlockSpec((bt, H), lambda i: (i, 0)),
            pl.BlockSpec((bt, H), lambda i: (i, 0)),
            pl.BlockSpec((bt, H), lambda i: (i, 0)),
            pl.BlockSpec((bt, 1), lambda i: (i, 0)),
            pl.BlockSpec((bt, 1), lambda i: (i, 0)),
        ],
        out_specs=pl.BlockSpec((bt, H), lambda i: (i, 0)),
        out_shape=jax.ShapeDtypeStruct((T, H), F32),
    )(base, y0g, y1g, w0.reshape(T, 1), w1.reshape(T, 1))


def _route(logits):
    probs = jax.nn.softmax(logits, axis=-1)
    topv, topi = jax.lax.top_k(probs, TOPK)
    topw = topv / jnp.sum(topv, axis=-1, keepdims=True)
    e_flat = topi.reshape(-1)
    order = jnp.argsort(e_flat, stable=True)
    sorted_e = e_flat[order]
    counts = jnp.bincount(e_flat, length=E)
    excl = jnp.cumsum(counts) - counts
    padded = ((counts + BT - 1) // BT) * BT
    poff = jnp.cumsum(padded) - padded
    within = jnp.arange(P, dtype=jnp.int32) - excl[sorted_e]
    dest = poff[sorted_e] + within
    tok_pad = jnp.zeros((P_PAD,), jnp.int32).at[dest].set(
        (order // TOPK).astype(jnp.int32))
    tile_start = jnp.arange(NT, dtype=jnp.int32) * BT
    pend = poff + padded
    nv = (jnp.sum(padded) // BT).astype(jnp.int32).reshape(1)
    tile_e_raw = jnp.minimum(
        jnp.sum(tile_start[:, None] >= pend[None, :], axis=1), E - 1
    ).astype(jnp.int32)
    tile_e = tile_e_raw[jnp.minimum(jnp.arange(NT, dtype=jnp.int32), nv[0] - 1)]
    pos = jnp.zeros((P,), jnp.int32).at[order].set(dest.astype(jnp.int32))
    pos2 = pos.reshape(T, TOPK)
    return tok_pad, tile_e, nv, pos2, topw


def kernel(hidden_states, positions, kv_cache, slot_mapping, seq_lens,
           ln1_w, ln2_w, wq, wkv, wo, gate_w, w1, w2, shexp_w, shdown_w):
    x = hidden_states.reshape(T, H)

    wq_nope = wq.reshape(HEADS, QHD, H)[:, :NOPE, :].reshape(HEADS * NOPE, H)
    w_qkv = jnp.concatenate(
        [wq_nope, wkv[:HEADS * (NOPE + VD)]], axis=0).astype(BF16)
    wo_bf = wo.astype(BF16)
    gate_bf = gate_w.astype(BF16)
    w1_bf = w1.astype(BF16)
    w2_bf = w2.astype(BF16)
    shexp_bf = shexp_w.astype(BF16)
    shdown_bf = shdown_w.astype(BF16)

    hn_bf = _rmsnorm_bf16(x, ln1_w)
    qkv = _qkv_proj(hn_bf, w_qkv)
    ctx = _attention(qkv)
    hidden = _out_proj(ctx, wo_bf, x)

    xm_bf, logits = _rmsnorm_gate(hidden, ln2_w, gate_bf)

    tok_pad, tile_e, nv, pos2, topw = _route(logits)

    xs_bf = xm_bf[tok_pad]
    y = _moe_grouped(xs_bf, w1_bf, w2_bf, tile_e, nv)
    y0g = y[pos2[:, 0]]
    y1g = y[pos2[:, 1]]

    act_sh = _shared_up(xm_bf, shexp_bf)
    base = _shared_down_base(act_sh, shdown_bf, hidden)

    out = _combine(base, y0g, y1g, topw[:, 0], topw[:, 1])
    return out.reshape(1, T, H)
